# baseline (speedup 1.0000x reference)
import functools

import jax
import jax.numpy as jnp
from jax import lax
from jax.experimental import pallas as pl
from jax.experimental.pallas import tpu as pltpu
from jax.experimental.pallas import tpu_sc as plsc

_L = 16
_UNROLL = 8
_G = 2
_SC_ROWS = 2048


def _rcumsum_rows(in_ref, out_ref, n):
    nblocks = n // (_L * _UNROLL)
    lane15 = jnp.full((_L, 1), _L - 1, dtype=jnp.int32)
    bcast_last = functools.partial(
        lax.gather,
        dimension_numbers=lax.GatherDimensionNumbers(
            offset_dims=(), collapsed_slice_dims=(0,), start_index_map=(0,)),
        slice_sizes=(1,),
        mode=lax.GatherScatterMode.PROMISE_IN_BOUNDS,
    )

    for ri in range(_G):
        def blk_body(b, carry_vec):
            base = (nblocks - 1 - b) * _UNROLL
            vs, cums, tots = [], [], []
            for u in range(_UNROLL):
                v = in_ref[ri, pl.ds((base + u) * _L, _L)]
                c = plsc.cumsum(v)
                vs.append(v)
                cums.append(c)
                tots.append(bcast_last(c, lane15))
            psuf = [None] * _UNROLL
            acc = None
            for u in reversed(range(_UNROLL)):
                psuf[u] = acc
                acc = tots[u] if acc is None else acc + tots[u]
            for u in range(_UNROLL):
                local = tots[u] - cums[u] + vs[u]
                if psuf[u] is not None:
                    local = local + psuf[u]
                out_ref[ri, pl.ds((base + u) * _L, _L)] = carry_vec + local
            return carry_vec + acc

        lax.fori_loop(0, nblocks, blk_body, jnp.zeros((_L,), jnp.float32))


def _sc_body(x_hbm, o_hbm, in_a, in_b, out_a, out_b, sia, sib, soa, sob,
             *, rows_per_worker, n):
    wid = lax.axis_index("s") * 2 + lax.axis_index("c")
    row0 = wid * rows_per_worker
    ngroups = rows_per_worker // _G
    npairs = ngroups // 2

    def copy_in(buf, sem, g):
        return pltpu.make_async_copy(
            x_hbm.at[pl.ds(row0 + g * _G, _G)], buf, sem)

    def copy_out(buf, sem, g):
        return pltpu.make_async_copy(
            buf, o_hbm.at[pl.ds(row0 + g * _G, _G)], sem)

    copy_in(in_a, sia, 0).start()

    def pair_body(k, _):
        ga = 2 * k
        copy_in(in_a, sia, ga).wait()
        copy_in(in_b, sib, ga + 1).start()

        @pl.when(k > 0)
        def _():
            copy_out(out_a, soa, ga - 2).wait()

        _rcumsum_rows(in_a, out_a, n)
        copy_out(out_a, soa, ga).start()

        copy_in(in_b, sib, ga + 1).wait()

        @pl.when(k < npairs - 1)
        def _():
            copy_in(in_a, sia, ga + 2).start()

        @pl.when(k > 0)
        def _():
            copy_out(out_b, sob, ga - 1).wait()

        _rcumsum_rows(in_b, out_b, n)
        copy_out(out_b, sob, ga + 1).start()
        return jnp.int32(0)

    lax.fori_loop(0, npairs, pair_body, jnp.int32(0))
    copy_out(out_a, soa, ngroups - 2).wait()
    copy_out(out_b, sob, ngroups - 1).wait()


def _sc_rcumsum(x, sc_rows):
    m, n = x.shape
    info = plsc.get_sparse_core_info()
    nw = info.num_cores * info.num_subcores
    rows_per_worker = sc_rows // nw
    mesh = plsc.VectorSubcoreMesh(core_axis_name="c", subcore_axis_name="s")
    buf = pltpu.VMEM((_G, n), jnp.float32)
    sc_k = functools.partial(
        pl.kernel,
        out_type=jax.ShapeDtypeStruct((sc_rows, n), x.dtype),
        mesh=mesh,
        scratch_types=[buf, buf, buf, buf,
                       pltpu.SemaphoreType.DMA, pltpu.SemaphoreType.DMA,
                       pltpu.SemaphoreType.DMA, pltpu.SemaphoreType.DMA],
        compiler_params=pltpu.CompilerParams(needs_layout_passes=False),
    )(functools.partial(_sc_body, rows_per_worker=rows_per_worker, n=n))
    return sc_k(x)


def _tc_kernel(x_ref, o_ref, carry_ref, *, W):
    j = pl.program_id(0)

    @pl.when(j == 0)
    def _():
        carry_ref[...] = jnp.zeros_like(carry_ref)

    x = x_ref[...]
    rows = jax.lax.broadcasted_iota(jnp.int32, (W, W), 0)
    cols = jax.lax.broadcasted_iota(jnp.int32, (W, W), 1)
    tri = (rows >= cols).astype(jnp.float32)
    rc = jax.lax.dot_general(
        x, tri, (((1,), (0,)), ((), ())),
        preferred_element_type=jnp.float32,
        precision=jax.lax.Precision.DEFAULT,
    )
    o_ref[...] = rc + carry_ref[...]
    carry_ref[...] = carry_ref[...] + rc[:, 0:1]


def _tc_rcumsum(x, row_start, rows):
    m, n = x.shape
    W = 512
    ncb = n // W
    rb = row_start // rows
    return pl.pallas_call(
        functools.partial(_tc_kernel, W=W),
        grid=(ncb,),
        in_specs=[pl.BlockSpec((rows, W), lambda j: (rb, ncb - 1 - j))],
        out_specs=pl.BlockSpec((rows, W), lambda j: (0, ncb - 1 - j)),
        out_shape=jax.ShapeDtypeStruct((rows, n), x.dtype),
        scratch_shapes=[pltpu.VMEM((rows, 1), jnp.float32)],
    )(x)


def kernel(x):
    m, _ = x.shape
    top = _sc_rcumsum(x, _SC_ROWS)
    bot = _tc_rcumsum(x, _SC_ROWS, m - _SC_ROWS)
    return jnp.concatenate([top, bot], axis=0)


# speedup vs baseline: 1.3887x; 1.3887x over previous
import functools

import jax
import jax.numpy as jnp
from jax import lax
from jax.experimental import pallas as pl
from jax.experimental.pallas import tpu as pltpu
from jax.experimental.pallas import tpu_sc as plsc

_L = 16
_UNROLL = 8
_G = 2


def _rcumsum_rows(in_ref, out_ref, n):
    nblocks = n // (_L * _UNROLL)
    lane15 = jnp.full((_L, 1), _L - 1, dtype=jnp.int32)
    bcast_last = functools.partial(
        lax.gather,
        dimension_numbers=lax.GatherDimensionNumbers(
            offset_dims=(), collapsed_slice_dims=(0,), start_index_map=(0,)),
        slice_sizes=(1,),
        mode=lax.GatherScatterMode.PROMISE_IN_BOUNDS,
    )

    for ri in range(_G):
        def blk_body(b, carry_vec):
            base = (nblocks - 1 - b) * _UNROLL
            vs, cums, tots = [], [], []
            for u in range(_UNROLL):
                v = in_ref[ri, pl.ds((base + u) * _L, _L)]
                c = plsc.cumsum(v)
                t = bcast_last(c, lane15)
                vs.append(v)
                cums.append(c)
                tots.append(t)
            psuf = [None] * _UNROLL
            acc = None
            for u in reversed(range(_UNROLL)):
                psuf[u] = acc
                acc = tots[u] if acc is None else acc + tots[u]
            for u in range(_UNROLL):
                local = tots[u] - cums[u] + vs[u]
                if psuf[u] is not None:
                    local = local + psuf[u]
                out_ref[ri, pl.ds((base + u) * _L, _L)] = carry_vec + local
            return carry_vec + acc

        lax.fori_loop(0, nblocks, blk_body, jnp.zeros((_L,), jnp.float32))


def _sc_rcumsum(x_hbm, o_hbm, in_a, in_b, out_a, out_b, sia, sib, soa, sob,
                *, rows_per_worker, n):
    wid = lax.axis_index("s") * 2 + lax.axis_index("c")
    row0 = wid * rows_per_worker
    ngroups = rows_per_worker // _G
    npairs = ngroups // 2

    def copy_in(buf, sem, g):
        return pltpu.make_async_copy(
            x_hbm.at[pl.ds(row0 + g * _G, _G)], buf, sem)

    def copy_out(buf, sem, g):
        return pltpu.make_async_copy(
            buf, o_hbm.at[pl.ds(row0 + g * _G, _G)], sem)

    copy_in(in_a, sia, 0).start()

    def pair_body(k, _):
        ga = 2 * k
        copy_in(in_a, sia, ga).wait()
        copy_in(in_b, sib, ga + 1).start()

        @pl.when(k > 0)
        def _():
            copy_out(out_a, soa, ga - 2).wait()

        _rcumsum_rows(in_a, out_a, n)
        copy_out(out_a, soa, ga).start()

        copy_in(in_b, sib, ga + 1).wait()

        @pl.when(k < npairs - 1)
        def _():
            copy_in(in_a, sia, ga + 2).start()

        @pl.when(k > 0)
        def _():
            copy_out(out_b, sob, ga - 1).wait()

        _rcumsum_rows(in_b, out_b, n)
        copy_out(out_b, sob, ga + 1).start()
        return jnp.int32(0)

    lax.fori_loop(0, npairs, pair_body, jnp.int32(0))
    copy_out(out_a, soa, ngroups - 2).wait()
    copy_out(out_b, sob, ngroups - 1).wait()


def kernel(x):
    m, n = x.shape
    info = plsc.get_sparse_core_info()
    nw = info.num_cores * info.num_subcores
    rows_per_worker = m // nw
    mesh = plsc.VectorSubcoreMesh(core_axis_name="c", subcore_axis_name="s")
    buf = pltpu.VMEM((_G, n), jnp.float32)
    sc_k = functools.partial(
        pl.kernel,
        out_type=jax.ShapeDtypeStruct((m, n), x.dtype),
        mesh=mesh,
        scratch_types=[buf, buf, buf, buf,
                       pltpu.SemaphoreType.DMA, pltpu.SemaphoreType.DMA,
                       pltpu.SemaphoreType.DMA, pltpu.SemaphoreType.DMA],
        compiler_params=pltpu.CompilerParams(needs_layout_passes=False),
    )(functools.partial(_sc_rcumsum, rows_per_worker=rows_per_worker, n=n))
    return sc_k(x)


# speedup vs baseline: 1.5187x; 1.0937x over previous
import functools

import jax
import jax.numpy as jnp
from jax import lax
from jax.experimental import pallas as pl
from jax.experimental.pallas import tpu as pltpu
from jax.experimental.pallas import tpu_sc as plsc

_L = 16
_UNROLL = 8
_G = 2
_SC_ROWS = 2048


def _rcumsum_rows(in_ref, out_ref, n):
    nblocks = n // (_L * _UNROLL)
    lane15 = jnp.full((_L, 1), _L - 1, dtype=jnp.int32)
    bcast_last = functools.partial(
        lax.gather,
        dimension_numbers=lax.GatherDimensionNumbers(
            offset_dims=(), collapsed_slice_dims=(0,), start_index_map=(0,)),
        slice_sizes=(1,),
        mode=lax.GatherScatterMode.PROMISE_IN_BOUNDS,
    )

    for ri in range(_G):
        def blk_body(b, carry_vec):
            base = (nblocks - 1 - b) * _UNROLL
            vs, cums, tots = [], [], []
            for u in range(_UNROLL):
                v = in_ref[ri, pl.ds((base + u) * _L, _L)]
                c = plsc.cumsum(v)
                vs.append(v)
                cums.append(c)
                tots.append(bcast_last(c, lane15))
            psuf = [None] * _UNROLL
            acc = None
            for u in reversed(range(_UNROLL)):
                psuf[u] = acc
                acc = tots[u] if acc is None else acc + tots[u]
            for u in range(_UNROLL):
                local = tots[u] - cums[u] + vs[u]
                if psuf[u] is not None:
                    local = local + psuf[u]
                out_ref[ri, pl.ds((base + u) * _L, _L)] = carry_vec + local
            return carry_vec + acc

        lax.fori_loop(0, nblocks, blk_body, jnp.zeros((_L,), jnp.float32))


def _sc_body(x_hbm, o_hbm, in_a, in_b, out_a, out_b, sia, sib, soa, sob,
             *, rows_per_worker, n):
    wid = lax.axis_index("s") * 2 + lax.axis_index("c")
    row0 = wid * rows_per_worker
    ngroups = rows_per_worker // _G
    npairs = ngroups // 2

    def copy_in(buf, sem, g):
        return pltpu.make_async_copy(
            x_hbm.at[pl.ds(row0 + g * _G, _G)], buf, sem)

    def copy_out(buf, sem, g):
        return pltpu.make_async_copy(
            buf, o_hbm.at[pl.ds(row0 + g * _G, _G)], sem)

    copy_in(in_a, sia, 0).start()

    def pair_body(k, _):
        ga = 2 * k
        copy_in(in_a, sia, ga).wait()
        copy_in(in_b, sib, ga + 1).start()

        @pl.when(k > 0)
        def _():
            copy_out(out_a, soa, ga - 2).wait()

        _rcumsum_rows(in_a, out_a, n)
        copy_out(out_a, soa, ga).start()

        copy_in(in_b, sib, ga + 1).wait()

        @pl.when(k < npairs - 1)
        def _():
            copy_in(in_a, sia, ga + 2).start()

        @pl.when(k > 0)
        def _():
            copy_out(out_b, sob, ga - 1).wait()

        _rcumsum_rows(in_b, out_b, n)
        copy_out(out_b, sob, ga + 1).start()
        return jnp.int32(0)

    lax.fori_loop(0, npairs, pair_body, jnp.int32(0))
    copy_out(out_a, soa, ngroups - 2).wait()
    copy_out(out_b, sob, ngroups - 1).wait()


def _sc_rcumsum(x, sc_rows):
    m, n = x.shape
    info = plsc.get_sparse_core_info()
    nw = info.num_cores * info.num_subcores
    rows_per_worker = sc_rows // nw
    mesh = plsc.VectorSubcoreMesh(core_axis_name="c", subcore_axis_name="s")
    buf = pltpu.VMEM((_G, n), jnp.float32)
    sc_k = functools.partial(
        pl.kernel,
        out_type=jax.ShapeDtypeStruct((m, n), x.dtype),
        mesh=mesh,
        scratch_types=[buf, buf, buf, buf,
                       pltpu.SemaphoreType.DMA, pltpu.SemaphoreType.DMA,
                       pltpu.SemaphoreType.DMA, pltpu.SemaphoreType.DMA],
        compiler_params=pltpu.CompilerParams(needs_layout_passes=False),
    )(functools.partial(_sc_body, rows_per_worker=rows_per_worker, n=n))
    return sc_k(x)


def _tc_kernel(x_ref, part_ref, o_ref, carry_ref, *, W):
    del part_ref
    j = pl.program_id(0)

    @pl.when(j == 0)
    def _():
        carry_ref[...] = jnp.zeros_like(carry_ref)

    x = x_ref[...]
    rows = jax.lax.broadcasted_iota(jnp.int32, (W, W), 0)
    cols = jax.lax.broadcasted_iota(jnp.int32, (W, W), 1)
    tri = (rows >= cols).astype(jnp.float32)
    rc = jax.lax.dot_general(
        x, tri, (((1,), (0,)), ((), ())),
        preferred_element_type=jnp.float32,
        precision=jax.lax.Precision.DEFAULT,
    )
    o_ref[...] = rc + carry_ref[...]
    carry_ref[...] = carry_ref[...] + rc[:, 0:1]


def _tc_fill_bottom(x, partial, row_start):
    m, n = x.shape
    rows = m - row_start
    W = 512
    ncb = n // W
    rb = row_start // rows
    return pl.pallas_call(
        functools.partial(_tc_kernel, W=W),
        grid=(ncb,),
        in_specs=[
            pl.BlockSpec((rows, W), lambda j: (rb, ncb - 1 - j)),
            pl.BlockSpec(memory_space=pl.ANY),
        ],
        out_specs=pl.BlockSpec((rows, W), lambda j: (rb, ncb - 1 - j)),
        out_shape=jax.ShapeDtypeStruct((m, n), x.dtype),
        scratch_shapes=[pltpu.VMEM((rows, 1), jnp.float32)],
        input_output_aliases={1: 0},
    )(x, partial)


def kernel(x):
    partial = _sc_rcumsum(x, _SC_ROWS)
    return _tc_fill_bottom(x, partial, _SC_ROWS)
